# Initial kernel scaffold; baseline (speedup 1.0000x reference)
import jax
import jax.numpy as jnp
from jax.experimental import pallas as pl


def kernel(inputs, weight):
    raise NotImplementedError("write your pallas kernel here")


# speedup vs baseline: 1.4691x; 1.4691x over previous
import functools

import jax
import jax.numpy as jnp
from jax import lax
from jax.experimental import pallas as pl
from jax.experimental.pallas import tpu as pltpu
from jax.experimental.pallas import tpu_sc as plsc

BATCH = 16384
FIELDS = 26
EMBEDDING_SIZE = 32

NC = 2
NS = 16
NW = NC * NS

B = BATCH * FIELDS
D = EMBEDDING_SIZE
BPW = B // NW
CHUNK = 128
NCHUNK = BPW // CHUNK
assert BPW * NW == B and NCHUNK * CHUNK == BPW


def _gather_kernel(idx_hbm, w_hbm, out_hbm, idx_v, rows_v, gsem, osem):
    wid = lax.axis_index("s") * NC + lax.axis_index("c")
    pltpu.sync_copy(idx_hbm.at[wid], idx_v)

    def gather_chunk(j, slot):
        return pltpu.make_async_copy(
            w_hbm.at[idx_v.at[j]], rows_v.at[slot], gsem)

    def out_chunk(j, slot):
        return pltpu.make_async_copy(
            rows_v.at[slot], out_hbm.at[wid, j], osem)

    gather_chunk(0, 0).start()

    def body(j, _):
        slot = lax.rem(j, 2)
        nxt = 1 - slot
        gather_chunk(j, slot).wait()
        gather_chunk(j + 1, nxt).start()
        oc = out_chunk(j, slot)
        oc.start()
        oc.wait()
        return 0

    lax.fori_loop(0, NCHUNK - 1, body, 0)
    last = NCHUNK - 1
    slot = lax.rem(last, 2)
    gather_chunk(last, slot).wait()
    oc = out_chunk(last, slot)
    oc.start()
    oc.wait()


@jax.jit
def kernel(inputs, weight):
    idx = inputs.astype(jnp.int32).reshape(NW, NCHUNK, CHUNK)
    mesh = plsc.VectorSubcoreMesh(core_axis_name="c", subcore_axis_name="s")
    out = pl.kernel(
        _gather_kernel,
        out_type=jax.ShapeDtypeStruct((NW, NCHUNK, CHUNK, D), jnp.float32),
        mesh=mesh,
        scratch_types=[
            pltpu.VMEM((NCHUNK, CHUNK), jnp.int32),
            pltpu.VMEM((2, CHUNK, D), jnp.float32),
            pltpu.SemaphoreType.DMA,
            pltpu.SemaphoreType.DMA,
        ],
        compiler_params=pltpu.CompilerParams(use_tc_tiling_on_sc=False),
    )(idx, weight)
    return out.reshape(BATCH, FIELDS, D)


# speedup vs baseline: 1.5537x; 1.0575x over previous
import functools

import jax
import jax.numpy as jnp
from jax import lax
from jax.experimental import pallas as pl
from jax.experimental.pallas import tpu as pltpu
from jax.experimental.pallas import tpu_sc as plsc

BATCH = 16384
FIELDS = 26
EMBEDDING_SIZE = 32

NC = 2
NS = 16
NW = NC * NS

B = BATCH * FIELDS
D = EMBEDDING_SIZE
BPW = B // NW
CHUNK = 512
NCHUNK = BPW // CHUNK
assert BPW * NW == B and NCHUNK * CHUNK == BPW


def _gather_kernel(idx_hbm, w_hbm, out_hbm, idx_v, rows_v, gsem, osem):
    wid = lax.axis_index("s") * NC + lax.axis_index("c")
    pltpu.sync_copy(idx_hbm.at[wid], idx_v)

    def gather_chunk(j, slot):
        return pltpu.make_async_copy(
            w_hbm.at[idx_v.at[j]], rows_v.at[slot], gsem)

    def out_chunk(j, slot):
        return pltpu.make_async_copy(
            rows_v.at[slot], out_hbm.at[wid, j], osem)

    gather_chunk(0, 0).start()

    def body(j, _):
        slot = lax.rem(j, 2)
        nxt = 1 - slot
        gather_chunk(j, slot).wait()
        gather_chunk(j + 1, nxt).start()
        oc = out_chunk(j, slot)
        oc.start()
        oc.wait()
        return 0

    lax.fori_loop(0, NCHUNK - 1, body, 0)
    last = NCHUNK - 1
    slot = lax.rem(last, 2)
    gather_chunk(last, slot).wait()
    oc = out_chunk(last, slot)
    oc.start()
    oc.wait()


@jax.jit
def kernel(inputs, weight):
    idx = inputs.astype(jnp.int32).reshape(NW, NCHUNK, CHUNK)
    mesh = plsc.VectorSubcoreMesh(core_axis_name="c", subcore_axis_name="s")
    out = pl.kernel(
        _gather_kernel,
        out_type=jax.ShapeDtypeStruct((NW, NCHUNK, CHUNK, D), jnp.float32),
        mesh=mesh,
        scratch_types=[
            pltpu.VMEM((NCHUNK, CHUNK), jnp.int32),
            pltpu.VMEM((2, CHUNK, D), jnp.float32),
            pltpu.SemaphoreType.DMA,
            pltpu.SemaphoreType.DMA,
        ],
        compiler_params=pltpu.CompilerParams(use_tc_tiling_on_sc=False),
    )(idx, weight)
    return out.reshape(BATCH, FIELDS, D)


# speedup vs baseline: 1.5776x; 1.0154x over previous
import functools

import jax
import jax.numpy as jnp
from jax import lax
from jax.experimental import pallas as pl
from jax.experimental.pallas import tpu as pltpu
from jax.experimental.pallas import tpu_sc as plsc

BATCH = 16384
FIELDS = 26
EMBEDDING_SIZE = 32

NC = 2
NS = 16
NW = NC * NS

B = BATCH * FIELDS
D = EMBEDDING_SIZE
BPW = B // NW
CHUNK = 512
NCHUNK = BPW // CHUNK
assert BPW * NW == B and NCHUNK * CHUNK == BPW


NBUF = 4


def _gather_kernel(idx_hbm, w_hbm, out_hbm, idx_v, rows_v, gsem, osem):
    wid = lax.axis_index("s") * NC + lax.axis_index("c")
    pltpu.sync_copy(idx_hbm.at[wid], idx_v)

    def gather_chunk(j, slot):
        return pltpu.make_async_copy(
            w_hbm.at[idx_v.at[j]], rows_v.at[slot], gsem)

    def out_chunk(j, slot):
        return pltpu.make_async_copy(
            rows_v.at[slot], out_hbm.at[wid, j], osem)

    gather_chunk(0, 0).start()
    gather_chunk(1, 1).start()

    for j in (0, 1):
        gather_chunk(j, j).wait()
        gather_chunk(j + 2, j + 2).start()
        out_chunk(j, j).start()

    def body(j, _):
        slot = lax.rem(j, NBUF)
        gather_chunk(j, slot).wait()
        out_chunk(j - 2, lax.rem(j - 2, NBUF)).wait()
        gather_chunk(j + 2, lax.rem(j + 2, NBUF)).start()
        out_chunk(j, slot).start()
        return 0

    lax.fori_loop(2, NCHUNK - 2, body, 0)

    for j in (NCHUNK - 2, NCHUNK - 1):
        gather_chunk(j, j % NBUF).wait()
        out_chunk(j - 2, (j - 2) % NBUF).wait()
        out_chunk(j, j % NBUF).start()
    for j in (NCHUNK - 2, NCHUNK - 1):
        out_chunk(j, j % NBUF).wait()


@jax.jit
def kernel(inputs, weight):
    idx = inputs.astype(jnp.int32).reshape(NW, NCHUNK, CHUNK)
    mesh = plsc.VectorSubcoreMesh(core_axis_name="c", subcore_axis_name="s")
    out = pl.kernel(
        _gather_kernel,
        out_type=jax.ShapeDtypeStruct((NW, NCHUNK, CHUNK, D), jnp.float32),
        mesh=mesh,
        scratch_types=[
            pltpu.VMEM((NCHUNK, CHUNK), jnp.int32),
            pltpu.VMEM((2, CHUNK, D), jnp.float32),
            pltpu.SemaphoreType.DMA,
            pltpu.SemaphoreType.DMA,
        ],
        compiler_params=pltpu.CompilerParams(use_tc_tiling_on_sc=False),
    )(idx, weight)
    return out.reshape(BATCH, FIELDS, D)
